# baseline (speedup 1.0000x reference)
import functools

import jax
import jax.numpy as jnp
from jax import lax
from jax.experimental import pallas as pl
from jax.experimental.pallas import tpu as pltpu
from jax.experimental.pallas import tpu_sc as plsc

_B = 4096
_A = 100
_D = 128
_L = 16


@functools.cache
def _build(nc, ns):
    nw = nc * ns
    bpw = _B // nw
    mesh = plsc.VectorSubcoreMesh(core_axis_name="c", subcore_axis_name="s")

    @functools.partial(
        pl.kernel,
        mesh=mesh,
        out_type=jax.ShapeDtypeStruct((_B, _D), jnp.float32),
        scratch_types=[
            pltpu.VMEM((bpw * (_A + 1),), jnp.int32),
            pltpu.VMEM((_A * _D,), jnp.float32),
            pltpu.VMEM((bpw, _D), jnp.float32),
            pltpu.SemaphoreType.DMA,
            pltpu.SemaphoreType.DMA,
        ],
        compiler_params=pltpu.CompilerParams(needs_layout_passes=False),
    )
    def run(x_hbm, table_hbm, out_hbm, xv, tv, rows, sem, osem):
        wid = lax.axis_index("s") * nc + lax.axis_index("c")
        base = wid * bpw
        tbl_cp = pltpu.async_copy(table_hbm, tv, sem)
        pltpu.sync_copy(
            x_hbm.at[pl.ds(base * (_A + 1), bpw * (_A + 1))], xv)
        lane = lax.iota(jnp.int32, _L)
        zero = jnp.zeros((_L,), jnp.int32)
        pitch = _A + 1
        lane_off = lane * pitch
        tbl_cp.wait()
        ngroups = bpw // _L

        def rowsum(g):
            vec0 = lane_off + g * (_L * pitch)
            accs = [zero for _ in range(4)]
            for j in range(_A):
                accs[j % 4] = accs[j % 4] + plsc.load_gather(xv, [vec0 + j])
            return ((accs[0] + accs[1]) + (accs[2] + accs[3])) * _D

        def do_group(g, accd):
            nxt = rowsum(jnp.minimum(g + 1, ngroups - 1))
            for l in range(_L):
                r = g * _L + l
                src0 = lane + jnp.broadcast_to(accd[l], (_L,))
                for k in range(_D // _L):
                    rows[r, pl.ds(k * _L, _L)] = plsc.load_gather(
                        tv, [src0 + (k * _L)])
            pltpu.async_copy(rows.at[pl.ds(g * _L, _L)],
                             out_hbm.at[pl.ds(base + g * _L, _L)], osem)
            return nxt

        lax.fori_loop(0, ngroups, do_group, rowsum(0))
        pltpu.make_async_copy(out_hbm.at[pl.ds(base, bpw)], rows, osem).wait()

    return run


def kernel(x, table):
    info = plsc.get_sparse_core_info()
    run = _build(info.num_cores, info.num_subcores)
    table_flat = table.reshape(_A * _D)
    x_pad = jnp.pad(x.astype(jnp.int32), ((0, 0), (0, 1))).reshape(-1)
    return run(x_pad, table_flat)


# speedup vs baseline: 1.0615x; 1.0615x over previous
import functools

import jax
import jax.numpy as jnp
from jax import lax
from jax.experimental import pallas as pl
from jax.experimental.pallas import tpu as pltpu
from jax.experimental.pallas import tpu_sc as plsc

_B = 4096
_A = 100
_D = 128
_L = 16


@functools.cache
def _build(nc, ns):
    nw = nc * ns
    bpw = _B // nw
    mesh = plsc.VectorSubcoreMesh(core_axis_name="c", subcore_axis_name="s")

    @functools.partial(
        pl.kernel,
        mesh=mesh,
        out_type=jax.ShapeDtypeStruct((_B, _D), jnp.float32),
        scratch_types=[
            pltpu.VMEM((bpw * (_A + 1),), jnp.int32),
            pltpu.VMEM((_A * _D,), jnp.float32),
            pltpu.VMEM((bpw, _D), jnp.float32),
            pltpu.SemaphoreType.DMA,
            pltpu.SemaphoreType.DMA,
        ],
        compiler_params=pltpu.CompilerParams(needs_layout_passes=False),
    )
    def run(x_hbm, table_hbm, out_hbm, xv, tv, rows, sem, osem):
        wid = lax.axis_index("s") * nc + lax.axis_index("c")
        base = wid * bpw
        tbl_cp = pltpu.async_copy(table_hbm, tv, sem)
        pltpu.sync_copy(
            x_hbm.at[pl.ds(base * (_A + 1), bpw * (_A + 1))], xv)
        lane = lax.iota(jnp.int32, _L)
        zero = jnp.zeros((_L,), jnp.int32)
        pitch = _A + 1
        lane_off = lane * pitch
        tbl_cp.wait()

        def do_group(g, _):
            vec0 = lane_off + g * (_L * pitch)
            accs = [zero for _ in range(4)]
            for j in range(_A):
                accs[j % 4] = accs[j % 4] + plsc.load_gather(xv, [vec0 + j])
            acc = (accs[0] + accs[1]) + (accs[2] + accs[3])
            accd = acc * _D
            for l in range(_L):
                r = g * _L + l
                src0 = lane + jnp.broadcast_to(accd[l], (_L,))
                for k in range(_D // _L):
                    rows[r, pl.ds(k * _L, _L)] = plsc.load_gather(
                        tv, [src0 + (k * _L)])
            pltpu.async_copy(rows.at[pl.ds(g * _L, _L)],
                             out_hbm.at[pl.ds(base + g * _L, _L)], osem)
            return 0

        lax.fori_loop(0, bpw // _L, do_group, 0)
        pltpu.make_async_copy(out_hbm.at[pl.ds(base, bpw)], rows, osem).wait()

    return run


def kernel(x, table):
    info = plsc.get_sparse_core_info()
    run = _build(info.num_cores, info.num_subcores)
    table_flat = table.reshape(_A * _D)
    x_pad = jnp.pad(x.astype(jnp.int32), ((0, 0), (0, 1))).reshape(-1)
    return run(x_pad, table_flat)
